# Initial kernel scaffold; baseline (speedup 1.0000x reference)
import jax
import jax.numpy as jnp
from jax.experimental import pallas as pl


def kernel(prefix, embedding_table):
    raise NotImplementedError("write your pallas kernel here")


# speedup vs baseline: 1.3725x; 1.3725x over previous
import functools

import jax
import jax.numpy as jnp
from jax import lax
from jax.experimental import pallas as pl
from jax.experimental.pallas import tpu as pltpu
from jax.experimental.pallas import tpu_sc as plsc

NC = 2
NS = 16
NW = NC * NS
CHUNK = 2


def _gather_body(n_chunks, b_per_w, table_hbm, idx_hbm, out_hbm,
                 idx_v, buf0, buf1, g0, g1, s0, s1):
    wid = lax.axis_index("s") * NC + lax.axis_index("c")
    base = wid * b_per_w
    bufs = (buf0, buf1)
    gsems = (g0, g1)
    ssems = (s0, s1)

    pltpu.sync_copy(idx_hbm.at[wid], idx_v)

    pltpu.async_copy(table_hbm.at[idx_v.at[0]], buf0, g0)
    pltpu.async_copy(table_hbm.at[idx_v.at[1]], buf1, g1)

    def step(g, b, start_next):
        pltpu.make_async_copy(table_hbm.at[pl.ds(0, CHUNK)], bufs[b],
                              gsems[b]).wait()
        pltpu.async_copy(bufs[b], out_hbm.at[pl.ds(base + g * CHUNK, CHUNK)],
                         ssems[b])
        pltpu.make_async_copy(table_hbm.at[pl.ds(0, CHUNK)], bufs[b],
                              ssems[b]).wait()
        if start_next:
            pltpu.async_copy(table_hbm.at[idx_v.at[g + 2]], bufs[b], gsems[b])

    @pl.loop(0, n_chunks - 2, step=2)
    def _(gp):
        for b in range(2):
            step(gp + b, b, True)

    for g in (n_chunks - 2, n_chunks - 1):
        step(g, g % 2, False)


def kernel(prefix, embedding_table):
    bsz, toks = prefix.shape
    vocab, dim = embedding_table.shape
    n_lookups = bsz * toks
    b_per_w = n_lookups // NW
    n_chunks = b_per_w // CHUNK

    idx = jnp.asarray(prefix, jnp.int32).reshape(NW, n_chunks, CHUNK)

    mesh = plsc.VectorSubcoreMesh(core_axis_name="c", subcore_axis_name="s")
    sc_gather = pl.kernel(
        functools.partial(_gather_body, n_chunks, b_per_w),
        out_type=jax.ShapeDtypeStruct((n_lookups, dim), jnp.float32),
        mesh=mesh,
        scratch_types=[
            pltpu.VMEM((n_chunks, CHUNK), jnp.int32),
            pltpu.VMEM((CHUNK, dim), jnp.float32),
            pltpu.VMEM((CHUNK, dim), jnp.float32),
            pltpu.SemaphoreType.DMA,
            pltpu.SemaphoreType.DMA,
            pltpu.SemaphoreType.DMA,
            pltpu.SemaphoreType.DMA,
        ],
    )
    out = sc_gather(embedding_table, idx)
    return out.reshape(bsz, toks, dim)
